# baseline (speedup 1.0000x reference)
import numpy as np
import jax
import jax.numpy as jnp
from jax import lax
from jax.experimental import pallas as pl
from jax.experimental.pallas import tpu as pltpu
from jax.experimental.pallas import tpu_sc as plsc

_NUM_LEVELS = 16
_T = 2 ** 14
_BASE_RES = 16
_SCALE = float(np.exp2(np.log2(1024 / 16) / (_NUM_LEVELS - 1)))
_N = 524288
_P2 = int(np.array([2654435761], np.uint32).view(np.int32)[0])
_P3 = 805459861

_RES = [int(np.floor(_BASE_RES * _SCALE ** l)) for l in range(_NUM_LEVELS)]
_NDENSE = sum(1 for r in _RES if (r + 1) ** 3 <= _T)

_NC, _NS = 2, 16
_B = 4096
_NBLK = _N // _B


def _sc_body(xb3, embf, out, tab0, tab1, tab2, tab3,
             xbuf0, xbuf1, fbuf0, fbuf1, sin0, sin1, sout0, sout1):
    c = lax.axis_index("c")
    level = lax.axis_index("s")
    ebase = level * (4 * _T)
    pltpu.sync_copy(embf.at[pl.ds(ebase + 0 * _T, _T)], tab0)
    pltpu.sync_copy(embf.at[pl.ds(ebase + 1 * _T, _T)], tab1)
    pltpu.sync_copy(embf.at[pl.ds(ebase + 2 * _T, _T)], tab2)
    pltpu.sync_copy(embf.at[pl.ds(ebase + 3 * _T, _T)], tab3)

    nblk = _NBLK // _NC
    gbase = c * nblk
    xbufs = (xbuf0, xbuf1)
    fbufs = (fbuf0, fbuf1)
    sins = (sin0, sin1)
    souts = (sout0, sout1)

    def start_in(g, sl):
        pltpu.make_async_copy(xb3.at[pl.ds(g * (3 * _B), 3 * _B)],
                              xbufs[sl], sins[sl]).start()

    def wait_in(sl):
        pltpu.make_async_copy(xb3.at[pl.ds(0, 3 * _B)],
                              xbufs[sl], sins[sl]).wait()

    def start_out(g, sl):
        dst = out.at[pl.ds(g * (4 * _NUM_LEVELS * _B) + (level * 4) * _B,
                           4 * _B)]
        pltpu.make_async_copy(fbufs[sl], dst, souts[sl]).start()

    def wait_out(sl):
        dst = out.at[pl.ds((level * 4) * _B, 4 * _B)]
        pltpu.make_async_copy(fbufs[sl], dst, souts[sl]).wait()

    lane3 = lax.iota(jnp.int32, 16) * 3

    def compute(xb, fb, my, mz, resf, dense_path):
        @plsc.parallel_loop(0, _B, step=16, unroll=4)
        def vstep(off):
            i0 = lane3 + off * 3
            px = plsc.load_gather(xb, [i0]) * resf
            py = plsc.load_gather(xb, [i0 + 1]) * resf
            pz = plsc.load_gather(xb, [i0 + 2]) * resf
            ix = px.astype(jnp.int32)
            iy = py.astype(jnp.int32)
            iz = pz.astype(jnp.int32)
            fx = px - ix.astype(jnp.float32)
            fy = py - iy.astype(jnp.float32)
            fz = pz - iz.astype(jnp.float32)
            wx = (1.0 - fx, fx)
            wy = (1.0 - fy, fy)
            wz = (1.0 - fz, fz)
            ax = (ix, ix + 1)
            ay0 = iy * my
            az0 = iz * mz
            ay = (ay0, ay0 + my)
            az = (az0, az0 + mz)
            ayz = [[None, None], [None, None]]
            for dy in (0, 1):
                for dz in (0, 1):
                    if dense_path:
                        ayz[dy][dz] = ay[dy] + az[dz]
                    else:
                        ayz[dy][dz] = ay[dy] ^ az[dz]
            acc = [None] * 4
            tabs = (tab0, tab1, tab2, tab3)
            for dx in (0, 1):
                for dy in (0, 1):
                    wxy = wx[dx] * wy[dy]
                    for dz in (0, 1):
                        if dense_path:
                            idx = ax[dx] + ayz[dy][dz]
                        else:
                            idx = (ax[dx] ^ ayz[dy][dz]) & jnp.int32(_T - 1)
                        w = wxy * wz[dz]
                        for k in range(4):
                            g = plsc.load_gather(tabs[k], [idx])
                            if acc[k] is None:
                                acc[k] = w * g
                            else:
                                acc[k] = acc[k] + w * g
            fb[pl.ds(0 * _B + off, 16)] = acc[0]
            fb[pl.ds(1 * _B + off, 16)] = acc[1]
            fb[pl.ds(2 * _B + off, 16)] = acc[2]
            fb[pl.ds(3 * _B + off, 16)] = acc[3]

    def pipeline(dense_path):
        if dense_path:
            resf_s = jnp.where(level == 0, jnp.float32(_RES[0]),
                               jnp.float32(_RES[1]))
            my_s = jnp.where(level == 0, jnp.int32(_RES[0] + 1),
                             jnp.int32(_RES[1] + 1))
            mz_s = jnp.where(level == 0, jnp.int32((_RES[0] + 1) ** 2),
                             jnp.int32((_RES[1] + 1) ** 2))
            my = jnp.full((16,), my_s, dtype=jnp.int32)
            mz = jnp.full((16,), mz_s, dtype=jnp.int32)
        else:
            resf_s = jnp.float32(_RES[_NUM_LEVELS - 1])
            for l in reversed(range(_NDENSE, _NUM_LEVELS - 1)):
                resf_s = jnp.where(level == l, jnp.float32(_RES[l]), resf_s)
            my = jnp.full((16,), _P2, dtype=jnp.int32)
            mz = jnp.full((16,), _P3, dtype=jnp.int32)
        resf = jnp.full((16,), resf_s, dtype=jnp.float32)

        start_in(gbase, 0)

        def outer(b2, carry):
            for sl in (0, 1):
                b = b2 * 2 + sl
                g = gbase + b
                wait_in(sl)

                @pl.when(b + 1 < nblk)
                def _():
                    start_in(g + 1, 1 - sl)

                @pl.when(b >= 2)
                def _():
                    wait_out(sl)

                compute(xbufs[sl], fbufs[sl], my, mz, resf, dense_path)
                start_out(g, sl)
            return carry

        lax.fori_loop(0, nblk // 2, outer, 0)
        wait_out(0)
        wait_out(1)

    lax.cond(level < _NDENSE,
             lambda: pipeline(True),
             lambda: pipeline(False))


def _encode(xb3f, embf):
    mesh = plsc.VectorSubcoreMesh(core_axis_name="c", subcore_axis_name="s",
                                  num_cores=_NC, num_subcores=_NS)
    f = pl.kernel(
        _sc_body,
        out_type=jax.ShapeDtypeStruct((_N * 4 * _NUM_LEVELS,), jnp.float32),
        mesh=mesh,
        compiler_params=pltpu.CompilerParams(needs_layout_passes=False),
        cost_estimate=pl.CostEstimate(flops=2_000_000_000,
                                      bytes_accessed=300_000_000,
                                      transcendentals=0),
        scratch_types=[
            pltpu.VMEM((_T,), jnp.float32),
            pltpu.VMEM((_T,), jnp.float32),
            pltpu.VMEM((_T,), jnp.float32),
            pltpu.VMEM((_T,), jnp.float32),
            pltpu.VMEM((3 * _B,), jnp.float32),
            pltpu.VMEM((3 * _B,), jnp.float32),
            pltpu.VMEM((4 * _B,), jnp.float32),
            pltpu.VMEM((4 * _B,), jnp.float32),
            pltpu.SemaphoreType.DMA,
            pltpu.SemaphoreType.DMA,
            pltpu.SemaphoreType.DMA,
            pltpu.SemaphoreType.DMA,
        ],
    )
    return f(xb3f, embf)


def _mlp_body(x_ref, hb_ref, a0x_ref, a0h_ref, a1_ref, a2_ref, out_ref):
    xb = x_ref[...] * 2.0 - 1.0
    dn = (((1,), (0,)), ((), ()))
    z0 = lax.dot_general(a0x_ref[...], xb, (((1,), (1,)), ((), ())),
                         preferred_element_type=jnp.float32)
    z0 = z0 + lax.dot_general(a0h_ref[...], hb_ref[0], dn,
                              preferred_element_type=jnp.float32)
    z0 = jnp.maximum(z0, 0.0)
    z1 = jnp.maximum(
        lax.dot_general(a1_ref[...], z0, dn,
                        preferred_element_type=jnp.float32), 0.0)
    out_ref[...] = jnp.abs(
        lax.dot_general(z1, a2_ref[...], (((0,), (0,)), ((), ())),
                        preferred_element_type=jnp.float32))


def _mlp(x, h3, a0x, a0h, a1, a2):
    grid = (_NBLK,)
    return pl.pallas_call(
        _mlp_body,
        grid=grid,
        in_specs=[
            pl.BlockSpec((_B, 3), lambda i: (i, 0)),
            pl.BlockSpec((1, 4 * _NUM_LEVELS, _B), lambda i: (i, 0, 0)),
            pl.BlockSpec((64, 3), lambda i: (0, 0)),
            pl.BlockSpec((64, 64), lambda i: (0, 0)),
            pl.BlockSpec((64, 64), lambda i: (0, 0)),
            pl.BlockSpec((64, 3), lambda i: (0, 0)),
        ],
        out_specs=pl.BlockSpec((_B, 3), lambda i: (i, 0)),
        out_shape=jax.ShapeDtypeStruct((_N, 3), jnp.float32),
    )(x, h3, a0x, a0h, a1, a2)


def kernel(x, embeddings, W0, W1, W2):
    embf = jnp.transpose(embeddings, (0, 2, 1)).reshape(-1)
    hflat = _encode(x.reshape(-1), embf)
    h3 = hflat.reshape(_NBLK, 4 * _NUM_LEVELS, _B)
    a0x = W0[:3].T
    a0h = W0[3:].T
    a1 = W1.T
    return _mlp(x, h3, a0x, a0h, a1, W2)


# speedup vs baseline: 1.2440x; 1.2440x over previous
import numpy as np
import jax
import jax.numpy as jnp
from jax import lax
from jax.experimental import pallas as pl
from jax.experimental.pallas import tpu as pltpu
from jax.experimental.pallas import tpu_sc as plsc

_NUM_LEVELS = 16
_T = 2 ** 14
_BASE_RES = 16
_SCALE = float(np.exp2(np.log2(1024 / 16) / (_NUM_LEVELS - 1)))
_N = 524288
_P2 = int(np.array([2654435761], np.uint32).view(np.int32)[0])
_P3 = 805459861

_RES = [int(np.floor(_BASE_RES * _SCALE ** l)) for l in range(_NUM_LEVELS)]
_NDENSE = sum(1 for r in _RES if (r + 1) ** 3 <= _T)

_NC, _NS = 2, 16
_B = 4096
_NBLK = _N // _B


def _sc_body(xb3, embf, out, tab0, tab1, tab2, tab3,
             xbuf0, xbuf1, fbuf0, fbuf1, sin0, sin1, sout0, sout1):
    c = lax.axis_index("c")
    level = lax.axis_index("s")
    ebase = level * (4 * _T)
    pltpu.sync_copy(embf.at[pl.ds(ebase + 0 * _T, _T)], tab0)
    pltpu.sync_copy(embf.at[pl.ds(ebase + 1 * _T, _T)], tab1)
    pltpu.sync_copy(embf.at[pl.ds(ebase + 2 * _T, _T)], tab2)
    pltpu.sync_copy(embf.at[pl.ds(ebase + 3 * _T, _T)], tab3)

    nblk = _NBLK // _NC
    gbase = c * nblk
    xbufs = (xbuf0, xbuf1)
    fbufs = (fbuf0, fbuf1)
    sins = (sin0, sin1)
    souts = (sout0, sout1)

    def start_in(g, sl):
        pltpu.make_async_copy(xb3.at[pl.ds(g * (3 * _B), 3 * _B)],
                              xbufs[sl], sins[sl]).start()

    def wait_in(sl):
        pltpu.make_async_copy(xb3.at[pl.ds(0, 3 * _B)],
                              xbufs[sl], sins[sl]).wait()

    def start_out(g, sl):
        dst = out.at[pl.ds(g * (4 * _NUM_LEVELS * _B) + (level * 4) * _B,
                           4 * _B)]
        pltpu.make_async_copy(fbufs[sl], dst, souts[sl]).start()

    def wait_out(sl):
        dst = out.at[pl.ds((level * 4) * _B, 4 * _B)]
        pltpu.make_async_copy(fbufs[sl], dst, souts[sl]).wait()

    lane3 = lax.iota(jnp.int32, 16) * 3

    def compute(xb, fb, my, mz, resf, dense_path):
        @plsc.parallel_loop(0, _B, step=16, unroll=4)
        def vstep(off):
            px = xb[pl.ds(off, 16)] * resf
            py = xb[pl.ds(_B + off, 16)] * resf
            pz = xb[pl.ds(2 * _B + off, 16)] * resf
            ix = px.astype(jnp.int32)
            iy = py.astype(jnp.int32)
            iz = pz.astype(jnp.int32)
            fx = px - ix.astype(jnp.float32)
            fy = py - iy.astype(jnp.float32)
            fz = pz - iz.astype(jnp.float32)
            wx = (1.0 - fx, fx)
            wy = (1.0 - fy, fy)
            wz = (1.0 - fz, fz)
            ax = (ix, ix + 1)
            ay0 = iy * my
            az0 = iz * mz
            ay = (ay0, ay0 + my)
            az = (az0, az0 + mz)
            ayz = [[None, None], [None, None]]
            for dy in (0, 1):
                for dz in (0, 1):
                    if dense_path:
                        ayz[dy][dz] = ay[dy] + az[dz]
                    else:
                        ayz[dy][dz] = ay[dy] ^ az[dz]
            acc = [None] * 4
            tabs = (tab0, tab1, tab2, tab3)
            for dx in (0, 1):
                for dy in (0, 1):
                    wxy = wx[dx] * wy[dy]
                    for dz in (0, 1):
                        if dense_path:
                            idx = ax[dx] + ayz[dy][dz]
                        else:
                            idx = (ax[dx] ^ ayz[dy][dz]) & jnp.int32(_T - 1)
                        w = wxy * wz[dz]
                        for k in range(4):
                            g = plsc.load_gather(tabs[k], [idx])
                            if acc[k] is None:
                                acc[k] = w * g
                            else:
                                acc[k] = acc[k] + w * g
            fb[pl.ds(0 * _B + off, 16)] = acc[0]
            fb[pl.ds(1 * _B + off, 16)] = acc[1]
            fb[pl.ds(2 * _B + off, 16)] = acc[2]
            fb[pl.ds(3 * _B + off, 16)] = acc[3]

    def pipeline(dense_path):
        if dense_path:
            resf_s = jnp.where(level == 0, jnp.float32(_RES[0]),
                               jnp.float32(_RES[1]))
            my_s = jnp.where(level == 0, jnp.int32(_RES[0] + 1),
                             jnp.int32(_RES[1] + 1))
            mz_s = jnp.where(level == 0, jnp.int32((_RES[0] + 1) ** 2),
                             jnp.int32((_RES[1] + 1) ** 2))
            my = jnp.full((16,), my_s, dtype=jnp.int32)
            mz = jnp.full((16,), mz_s, dtype=jnp.int32)
        else:
            resf_s = jnp.float32(_RES[_NUM_LEVELS - 1])
            for l in reversed(range(_NDENSE, _NUM_LEVELS - 1)):
                resf_s = jnp.where(level == l, jnp.float32(_RES[l]), resf_s)
            my = jnp.full((16,), _P2, dtype=jnp.int32)
            mz = jnp.full((16,), _P3, dtype=jnp.int32)
        resf = jnp.full((16,), resf_s, dtype=jnp.float32)

        start_in(gbase, 0)

        def outer(b2, carry):
            for sl in (0, 1):
                b = b2 * 2 + sl
                g = gbase + b
                wait_in(sl)

                @pl.when(b + 1 < nblk)
                def _():
                    start_in(g + 1, 1 - sl)

                @pl.when(b >= 2)
                def _():
                    wait_out(sl)

                compute(xbufs[sl], fbufs[sl], my, mz, resf, dense_path)
                start_out(g, sl)
            return carry

        lax.fori_loop(0, nblk // 2, outer, 0)
        wait_out(0)
        wait_out(1)

    lax.cond(level < _NDENSE,
             lambda: pipeline(True),
             lambda: pipeline(False))


def _encode(xb3f, embf):
    mesh = plsc.VectorSubcoreMesh(core_axis_name="c", subcore_axis_name="s",
                                  num_cores=_NC, num_subcores=_NS)
    f = pl.kernel(
        _sc_body,
        out_type=jax.ShapeDtypeStruct((_N * 4 * _NUM_LEVELS,), jnp.float32),
        mesh=mesh,
        compiler_params=pltpu.CompilerParams(needs_layout_passes=False),
        cost_estimate=pl.CostEstimate(flops=2_000_000_000,
                                      bytes_accessed=300_000_000,
                                      transcendentals=0),
        scratch_types=[
            pltpu.VMEM((_T,), jnp.float32),
            pltpu.VMEM((_T,), jnp.float32),
            pltpu.VMEM((_T,), jnp.float32),
            pltpu.VMEM((_T,), jnp.float32),
            pltpu.VMEM((3 * _B,), jnp.float32),
            pltpu.VMEM((3 * _B,), jnp.float32),
            pltpu.VMEM((4 * _B,), jnp.float32),
            pltpu.VMEM((4 * _B,), jnp.float32),
            pltpu.SemaphoreType.DMA,
            pltpu.SemaphoreType.DMA,
            pltpu.SemaphoreType.DMA,
            pltpu.SemaphoreType.DMA,
        ],
    )
    return f(xb3f, embf)


def _mlp_body(x_ref, hb_ref, a0x_ref, a0h_ref, a1_ref, a2_ref, out_ref):
    xb = x_ref[0] * 2.0 - 1.0
    dn = (((1,), (0,)), ((), ()))
    z0 = lax.dot_general(a0x_ref[...], xb, dn,
                         preferred_element_type=jnp.float32)
    z0 = z0 + lax.dot_general(a0h_ref[...], hb_ref[0], dn,
                              preferred_element_type=jnp.float32)
    z0 = jnp.maximum(z0, 0.0)
    z1 = jnp.maximum(
        lax.dot_general(a1_ref[...], z0, dn,
                        preferred_element_type=jnp.float32), 0.0)
    out_ref[...] = jnp.abs(
        lax.dot_general(z1, a2_ref[...], (((0,), (0,)), ((), ())),
                        preferred_element_type=jnp.float32))


def _mlp(xb3, h3, a0x, a0h, a1, a2):
    grid = (_NBLK,)
    return pl.pallas_call(
        _mlp_body,
        grid=grid,
        in_specs=[
            pl.BlockSpec((1, 3, _B), lambda i: (i, 0, 0)),
            pl.BlockSpec((1, 4 * _NUM_LEVELS, _B), lambda i: (i, 0, 0)),
            pl.BlockSpec((64, 3), lambda i: (0, 0)),
            pl.BlockSpec((64, 64), lambda i: (0, 0)),
            pl.BlockSpec((64, 64), lambda i: (0, 0)),
            pl.BlockSpec((64, 3), lambda i: (0, 0)),
        ],
        out_specs=pl.BlockSpec((_B, 3), lambda i: (i, 0)),
        out_shape=jax.ShapeDtypeStruct((_N, 3), jnp.float32),
    )(xb3, h3, a0x, a0h, a1, a2)


def kernel(x, embeddings, W0, W1, W2):
    embf = jnp.transpose(embeddings, (0, 2, 1)).reshape(-1)
    xb3 = x.reshape(_NBLK, _B, 3).transpose(0, 2, 1)
    hflat = _encode(xb3.reshape(-1), embf)
    h3 = hflat.reshape(_NBLK, 4 * _NUM_LEVELS, _B)
    a0x = W0[:3].T
    a0h = W0[3:].T
    a1 = W1.T
    return _mlp(xb3, h3, a0x, a0h, a1, W2)
